# Initial kernel scaffold; baseline (speedup 1.0000x reference)
import jax
import jax.numpy as jnp
from jax.experimental import pallas as pl


def kernel(coord, atype, spin, type_embed, W1, W2, W3):
    raise NotImplementedError("write your pallas kernel here")


# speedup vs baseline: 3.2466x; 3.2466x over previous
import functools

import jax
import jax.numpy as jnp
from jax.experimental import pallas as pl

_NTYPES_REAL = 2
_VIRTUAL_SCALE_MASK = (0.3, 0.35, 0.0, 0.0)


def _spin_block_kernel(coord_ref, spin_ref, atype_ref, w1_ref, te_ref,
                       w2_ref, w3_ref, out_real_ref, out_mag_ref, scale_ref):
    coord = coord_ref[...]
    spin = spin_ref[...]
    atype = atype_ref[...]

    ntypes = te_ref.shape[0]
    type_ids = jax.lax.broadcasted_iota(jnp.int32, (1, ntypes), 1)
    onehot_real = (atype == type_ids).astype(jnp.float32)
    onehot_virt = (atype + _NTYPES_REAL == type_ids).astype(jnp.float32)

    scale = sum((atype == t).astype(jnp.float32) * _VIRTUAL_SCALE_MASK[t]
                for t in range(len(_VIRTUAL_SCALE_MASK)))

    virt = coord + spin * scale

    w1 = w1_ref[...]
    te = te_ref[...]
    w2 = w2_ref[...]
    w3 = w3_ref[...]

    h_r = jnp.tanh(jnp.dot(coord, w1, preferred_element_type=jnp.float32)
                   + jnp.dot(onehot_real, te, preferred_element_type=jnp.float32))
    h_v = jnp.tanh(jnp.dot(virt, w1, preferred_element_type=jnp.float32)
                   + jnp.dot(onehot_virt, te, preferred_element_type=jnp.float32))
    h_r = jnp.tanh(jnp.dot(h_r, w2, preferred_element_type=jnp.float32))
    h_v = jnp.tanh(jnp.dot(h_v, w2, preferred_element_type=jnp.float32))
    o_r = jnp.dot(h_r, w3, preferred_element_type=jnp.float32)
    o_v = jnp.dot(h_v, w3, preferred_element_type=jnp.float32)

    out_real_ref[...] = o_r + o_v
    out_mag_ref[...] = o_v * scale
    scale_ref[...] = scale


@functools.partial(jax.jit, static_argnames=("block",))
def _run(coord2, spin2, atype2, W1, type_embed, W2, W3, block):
    n = coord2.shape[0]
    grid = (n // block,)
    full = lambda *s: pl.BlockSpec(s, lambda i: (0,) * len(s))
    out_real2, out_mag2, scale2 = pl.pallas_call(
        _spin_block_kernel,
        grid=grid,
        in_specs=[
            pl.BlockSpec((block, 3), lambda i: (i, 0)),
            pl.BlockSpec((block, 3), lambda i: (i, 0)),
            pl.BlockSpec((block, 1), lambda i: (i, 0)),
            full(*W1.shape),
            full(*type_embed.shape),
            full(*W2.shape),
            full(*W3.shape),
        ],
        out_specs=[
            pl.BlockSpec((block, 3), lambda i: (i, 0)),
            pl.BlockSpec((block, 3), lambda i: (i, 0)),
            pl.BlockSpec((block, 1), lambda i: (i, 0)),
        ],
        out_shape=[
            jax.ShapeDtypeStruct((n, 3), jnp.float32),
            jax.ShapeDtypeStruct((n, 3), jnp.float32),
            jax.ShapeDtypeStruct((n, 1), jnp.float32),
        ],
    )(coord2, spin2, atype2, W1, type_embed, W2, W3)
    return out_real2, out_mag2, scale2


def kernel(coord, atype, spin, type_embed, W1, W2, W3):
    nframes, nloc = atype.shape
    coord2 = coord.reshape(nframes * nloc, 3)
    spin2 = spin.reshape(nframes * nloc, 3)
    atype2 = atype.reshape(nframes * nloc, 1).astype(jnp.int32)
    out_real2, out_mag2, scale2 = _run(
        coord2, spin2, atype2, W1, type_embed, W2, W3, block=2048)
    out_real = out_real2.reshape(nframes, nloc, 3)
    out_mag = out_mag2.reshape(nframes, nloc, 3)
    mask = (scale2 > 0.0).reshape(nframes, nloc, 1)
    return out_real, out_mag, mask
